# baseline (speedup 1.0000x reference)
import functools

import jax
import jax.numpy as jnp
from jax import lax
from jax.experimental import pallas as pl
from jax.experimental.pallas import tpu as pltpu
from jax.experimental.pallas import tpu_sc as plsc

_HIDDEN = 64
_BATCH = 16384
_HIST = 50
_TOTAL = _BATCH * (1 + _HIST)

_info = plsc.get_sparse_core_info()
_NC, _NS = _info.num_cores, _info.num_subcores
_NW = _NC * _NS
_PER_W = _TOTAL // _NW
_CHUNK = 128
_NCHUNK = _PER_W // _CHUNK


def _make_gather():
    mesh = plsc.VectorSubcoreMesh(core_axis_name="c", subcore_axis_name="s")

    @functools.partial(
        pl.kernel,
        mesh=mesh,
        out_type=jax.ShapeDtypeStruct((_TOTAL, _HIDDEN), jnp.float32),
        scratch_types=[
            pltpu.VMEM((_CHUNK,), jnp.int32),
            pltpu.VMEM((_CHUNK, _HIDDEN), jnp.float32),
            pltpu.SemaphoreType.DMA,
        ],
        compiler_params=pltpu.CompilerParams(use_tc_tiling_on_sc=False),
    )
    def gather_kernel(idx_hbm, table_hbm, out_hbm, idx_v, rows_v, sem):
        wid = lax.axis_index("s") * _NC + lax.axis_index("c")
        base = wid * _PER_W

        def body(c, carry):
            off = base + c * _CHUNK
            pltpu.sync_copy(idx_hbm.at[pl.ds(off, _CHUNK)], idx_v)
            pltpu.async_copy(table_hbm.at[idx_v], rows_v, sem).wait()
            pltpu.sync_copy(rows_v, out_hbm.at[pl.ds(off, _CHUNK)])
            return carry

        lax.fori_loop(0, _NCHUNK, body, 0)

    return gather_kernel


_gather = _make_gather()


def kernel(tokens_a, tokens_b, embedding):
    flat = jnp.concatenate(
        [tokens_a.reshape(-1), tokens_b.reshape(-1)]).astype(jnp.int32)
    out = _gather(flat, embedding)
    out_a = out[:_BATCH]
    out_b = out[_BATCH:].reshape(_BATCH, _HIST, _HIDDEN)
    return (out_a, out_b)


# speedup vs baseline: 1.1246x; 1.1246x over previous
import functools

import jax
import jax.numpy as jnp
from jax import lax
from jax.experimental import pallas as pl
from jax.experimental.pallas import tpu as pltpu
from jax.experimental.pallas import tpu_sc as plsc

_HIDDEN = 64
_BATCH = 16384
_HIST = 50
_TOTAL = _BATCH * (1 + _HIST)

_info = plsc.get_sparse_core_info()
_NC, _NS = _info.num_cores, _info.num_subcores
_NW = _NC * _NS
_PER_W = _TOTAL // _NW
_CHUNK = 128
_NCHUNK = _PER_W // _CHUNK
_RING = 12
_DEPTH = 6
_NGROUP = _NCHUNK // _RING


def _make_gather():
    mesh = plsc.VectorSubcoreMesh(core_axis_name="c", subcore_axis_name="s")

    @functools.partial(
        pl.kernel,
        mesh=mesh,
        out_type=jax.ShapeDtypeStruct((_TOTAL, _HIDDEN), jnp.float32),
        scratch_types=[
            pltpu.VMEM((_NCHUNK, _CHUNK), jnp.int32),
            pltpu.VMEM((_RING, _CHUNK, _HIDDEN), jnp.float32),
            pltpu.SemaphoreType.DMA,
            pltpu.SemaphoreType.DMA,
        ],
        compiler_params=pltpu.CompilerParams(use_tc_tiling_on_sc=False),
    )
    def gather_kernel(idx_hbm, table_hbm, out_hbm, idx_v, rows_v,
                      gsem, osem):
        wid = lax.axis_index("s") * _NC + lax.axis_index("c")
        base = wid * _PER_W

        pltpu.sync_copy(idx_hbm.at[wid], idx_v)

        def start_gather(c, slot):
            pltpu.async_copy(table_hbm.at[idx_v.at[c]], rows_v.at[slot],
                             gsem)

        for b in range(_DEPTH):
            start_gather(b, b)

        def body(g, carry):
            for b in range(_RING):
                c = g * _RING + b
                pltpu.make_async_copy(
                    table_hbm.at[idx_v.at[0]], rows_v.at[b], gsem).wait()
                pltpu.async_copy(
                    rows_v.at[b],
                    out_hbm.at[pl.ds(base + c * _CHUNK, _CHUNK)], osem)
                @pl.when(c >= _RING - _DEPTH)
                def _():
                    pltpu.make_async_copy(
                        rows_v.at[0],
                        out_hbm.at[pl.ds(base, _CHUNK)], osem).wait()
                @pl.when(c + _DEPTH < _NCHUNK)
                def _():
                    start_gather(c + _DEPTH, (b + _DEPTH) % _RING)
            return carry

        lax.fori_loop(0, _NGROUP, body, 0)

        for _ in range(_RING - _DEPTH):
            pltpu.make_async_copy(
                rows_v.at[0], out_hbm.at[pl.ds(base, _CHUNK)], osem).wait()

    return gather_kernel


_gather = _make_gather()


def kernel(tokens_a, tokens_b, embedding):
    flat = jnp.concatenate(
        [tokens_a.reshape(-1), tokens_b.reshape(-1)]).astype(jnp.int32)
    idx = flat.reshape(_NW, _NCHUNK, _CHUNK)
    out = _gather(idx, embedding)
    out_a = out[:_BATCH]
    out_b = out[_BATCH:].reshape(_BATCH, _HIST, _HIDDEN)
    return (out_a, out_b)


# speedup vs baseline: 1.7149x; 1.5249x over previous
import functools

import jax
import jax.numpy as jnp
from jax import lax
from jax.experimental import pallas as pl
from jax.experimental.pallas import tpu as pltpu
from jax.experimental.pallas import tpu_sc as plsc

_HIDDEN = 64
_BATCH = 16384
_HIST = 50

_info = plsc.get_sparse_core_info()
_NC, _NS = _info.num_cores, _info.num_subcores
_NW = _NC * _NS
_A_PER_W = _BATCH // _NW
_ACHUNK = 128
_NACHUNK = _A_PER_W // _ACHUNK
_B_PER_W = _BATCH // _NW
_RING = 16
_DEPTH = 8
_NGROUP = _B_PER_W // _RING


def _make_gather():
    mesh = plsc.VectorSubcoreMesh(core_axis_name="c", subcore_axis_name="s")

    @functools.partial(
        pl.kernel,
        mesh=mesh,
        out_type=(
            jax.ShapeDtypeStruct((_BATCH, _HIDDEN), jnp.float32),
            jax.ShapeDtypeStruct((_BATCH, _HIST, _HIDDEN), jnp.float32),
        ),
        scratch_types=[
            pltpu.VMEM((_A_PER_W,), jnp.int32),
            pltpu.VMEM((_NACHUNK, _ACHUNK, _HIDDEN), jnp.float32),
            pltpu.VMEM((_B_PER_W, _HIST), jnp.int32),
            pltpu.VMEM((_RING, _HIST, _HIDDEN), jnp.float32),
            pltpu.SemaphoreType.DMA,
            pltpu.SemaphoreType.DMA,
        ],
        compiler_params=pltpu.CompilerParams(use_tc_tiling_on_sc=False),
    )
    def gather_kernel(ta_hbm, tb_hbm, table_hbm, outa_hbm, outb_hbm,
                      idx_a, arows, idx_b, brows, gsem, osem):
        wid = lax.axis_index("s") * _NC + lax.axis_index("c")
        abase = wid * _A_PER_W
        bbase = wid * _B_PER_W

        pltpu.sync_copy(ta_hbm.at[pl.ds(abase, _A_PER_W)], idx_a)
        for c in range(_NACHUNK):
            pltpu.async_copy(
                table_hbm.at[idx_a.at[pl.ds(c * _ACHUNK, _ACHUNK)]],
                arows.at[c], gsem)
        for c in range(_NACHUNK):
            pltpu.make_async_copy(
                table_hbm.at[idx_a.at[pl.ds(0, _ACHUNK)]], arows.at[c],
                gsem).wait()
            pltpu.async_copy(
                arows.at[c],
                outa_hbm.at[pl.ds(abase + c * _ACHUNK, _ACHUNK)], osem)
        for c in range(_NACHUNK):
            pltpu.make_async_copy(
                arows.at[0], outa_hbm.at[pl.ds(0, _ACHUNK)], osem).wait()

        pltpu.sync_copy(tb_hbm.at[pl.ds(bbase, _B_PER_W)], idx_b)

        def start_gather(r, slot):
            pltpu.async_copy(table_hbm.at[idx_b.at[r]], brows.at[slot],
                             gsem)

        for b in range(_DEPTH):
            start_gather(b, b)

        def body(g, carry):
            for b in range(_RING):
                r = g * _RING + b
                pltpu.make_async_copy(
                    table_hbm.at[idx_b.at[0]], brows.at[b], gsem).wait()
                pltpu.async_copy(brows.at[b], outb_hbm.at[bbase + r],
                                 osem)
                @pl.when(r >= _RING - _DEPTH)
                def _():
                    pltpu.make_async_copy(
                        brows.at[0], outb_hbm.at[0], osem).wait()
                @pl.when(r + _DEPTH < _B_PER_W)
                def _():
                    start_gather(r + _DEPTH, (b + _DEPTH) % _RING)
            return carry

        lax.fori_loop(0, _NGROUP, body, 0)

        for _ in range(_RING - _DEPTH):
            pltpu.make_async_copy(brows.at[0], outb_hbm.at[0],
                                  osem).wait()

    return gather_kernel


_gather = _make_gather()


def kernel(tokens_a, tokens_b, embedding):
    return _gather(tokens_a.astype(jnp.int32), tokens_b.astype(jnp.int32),
                   embedding)


# speedup vs baseline: 2.2929x; 1.3371x over previous
import functools

import jax
import jax.numpy as jnp
from jax import lax
from jax.experimental import pallas as pl
from jax.experimental.pallas import tpu as pltpu
from jax.experimental.pallas import tpu_sc as plsc

_HIDDEN = 64
_BATCH = 16384
_HIST = 50
_HIST_PAD = 56
_HPAD = 128

_info = plsc.get_sparse_core_info()
_NC, _NS = _info.num_cores, _info.num_subcores
_NW = _NC * _NS
_A_PER_W = _BATCH // _NW
_ACHUNK = 128
_NACHUNK = _A_PER_W // _ACHUNK
_B_PER_W = _BATCH // _NW
_RING = 16
_DEPTH = 8
_NGROUP = _B_PER_W // _RING


def _make_gather():
    mesh = plsc.VectorSubcoreMesh(core_axis_name="c", subcore_axis_name="s")

    @functools.partial(
        pl.kernel,
        mesh=mesh,
        out_type=(
            jax.ShapeDtypeStruct((_BATCH, _HIDDEN), jnp.float32),
            jax.ShapeDtypeStruct((_BATCH, _HIST_PAD, _HPAD), jnp.float32),
        ),
        scratch_types=[
            pltpu.VMEM((_A_PER_W,), jnp.int32),
            pltpu.VMEM((_NACHUNK, _ACHUNK, _HIDDEN), jnp.float32),
            pltpu.VMEM((_B_PER_W, _HIST), jnp.int32),
            pltpu.VMEM((_RING, _HIST, _HIDDEN), jnp.float32),
            pltpu.SemaphoreType.DMA,
            pltpu.SemaphoreType.DMA,
        ],
        compiler_params=pltpu.CompilerParams(use_tc_tiling_on_sc=False),
    )
    def gather_kernel(ta_hbm, tb_hbm, table_hbm, outa_hbm, outb_hbm,
                      idx_a, arows, idx_b, brows, gsem, osem):
        wid = lax.axis_index("s") * _NC + lax.axis_index("c")
        abase = wid * _A_PER_W
        bbase = wid * _B_PER_W

        pltpu.sync_copy(ta_hbm.at[pl.ds(abase, _A_PER_W)], idx_a)
        for c in range(_NACHUNK):
            pltpu.async_copy(
                table_hbm.at[idx_a.at[pl.ds(c * _ACHUNK, _ACHUNK)]],
                arows.at[c], gsem)
        for c in range(_NACHUNK):
            pltpu.make_async_copy(
                table_hbm.at[idx_a.at[pl.ds(0, _ACHUNK)]], arows.at[c],
                gsem).wait()
            pltpu.async_copy(
                arows.at[c],
                outa_hbm.at[pl.ds(abase + c * _ACHUNK, _ACHUNK)], osem)
        for c in range(_NACHUNK):
            pltpu.make_async_copy(
                arows.at[0], outa_hbm.at[pl.ds(0, _ACHUNK)], osem).wait()

        pltpu.sync_copy(tb_hbm.at[pl.ds(bbase, _B_PER_W)], idx_b)

        def start_gather(r, slot):
            pltpu.async_copy(table_hbm.at[idx_b.at[r]], brows.at[slot],
                             gsem)

        for b in range(_DEPTH):
            start_gather(b, b)

        def body(g, carry):
            for b in range(_RING):
                r = g * _RING + b
                pltpu.make_async_copy(
                    table_hbm.at[idx_b.at[0]], brows.at[b], gsem).wait()
                pltpu.async_copy(
                    brows.at[b],
                    outb_hbm.at[bbase + r, pl.ds(0, _HIST),
                                pl.ds(0, _HIDDEN)], osem)
                @pl.when(r >= _RING - _DEPTH)
                def _():
                    pltpu.make_async_copy(
                        brows.at[0],
                        outb_hbm.at[0, pl.ds(0, _HIST), pl.ds(0, _HIDDEN)],
                        osem).wait()
                @pl.when(r + _DEPTH < _B_PER_W)
                def _():
                    start_gather(r + _DEPTH, (b + _DEPTH) % _RING)
            return carry

        lax.fori_loop(0, _NGROUP, body, 0)

        for _ in range(_RING - _DEPTH):
            pltpu.make_async_copy(
                brows.at[0],
                outb_hbm.at[0, pl.ds(0, _HIST), pl.ds(0, _HIDDEN)],
                osem).wait()

    return gather_kernel


_gather = _make_gather()


def kernel(tokens_a, tokens_b, embedding):
    out_a, out_b_pad = _gather(tokens_a.astype(jnp.int32),
                               tokens_b.astype(jnp.int32), embedding)
    return (out_a, out_b_pad[:, :_HIST, :_HIDDEN])
